# baseline (speedup 1.0000x reference)
import functools

import jax
import jax.numpy as jnp
from jax import lax
from jax.experimental import pallas as pl
from jax.experimental.pallas import tpu as pltpu
from jax.experimental.pallas import tpu_sc as plsc

VOCAB = 128100
HIDDEN = 768
MAX_POS = 8192
NUM_TOKENS = 16384
EPS = 1e-7

NC = 2
NS = 16
NW = NC * NS
L = 16
TOK_PER_W = NUM_TOKENS // NW
C = 64
NCHUNK = TOK_PER_W // C
NSL = HIDDEN // L


def _vrsqrt(v):
    i = lax.bitcast_convert_type(v, jnp.int32)
    i = jnp.int32(0x5F3759DF) - lax.shift_right_logical(i, jnp.int32(1))
    y = lax.bitcast_convert_type(i, jnp.float32)
    half = v * jnp.float32(0.5)
    for _ in range(3):
        y = y * (jnp.float32(1.5) - half * y * y)
    return y


_mesh = plsc.VectorSubcoreMesh(core_axis_name="c", subcore_axis_name="s")


@functools.partial(
    pl.kernel,
    mesh=_mesh,
    out_type=jax.ShapeDtypeStruct((NUM_TOKENS, HIDDEN), jnp.float32),
    compiler_params=pltpu.CompilerParams(use_tc_tiling_on_sc=False,
                                         needs_layout_passes=False),
    scratch_types=[
        pltpu.VMEM((TOK_PER_W,), jnp.int32),
        pltpu.VMEM((TOK_PER_W,), jnp.int32),
        pltpu.VMEM((C, HIDDEN), jnp.float32),
        pltpu.VMEM((C, HIDDEN), jnp.float32),
        pltpu.VMEM((HIDDEN,), jnp.float32),
        pltpu.VMEM((HIDDEN,), jnp.float32),
        pltpu.SemaphoreType.DMA,
    ],
)
def _emb_ln_kernel(ids_hbm, pids_hbm, wtab_hbm, ptab_hbm, g_hbm, b_hbm,
                   out_hbm, widx, pidx, wbuf, pbuf, gv, bv, sem):
    wid = lax.axis_index("s") * NC + lax.axis_index("c")
    base = wid * TOK_PER_W

    pltpu.sync_copy(ids_hbm.at[pl.ds(base, TOK_PER_W)], widx)
    pltpu.sync_copy(pids_hbm.at[pl.ds(base, TOK_PER_W)], pidx)
    pltpu.sync_copy(g_hbm, gv)
    pltpu.sync_copy(b_hbm, bv)

    def chunk_body(i, carry):
        pltpu.async_copy(wtab_hbm.at[widx.at[pl.ds(i * C, C)]], wbuf, sem).wait()
        pltpu.async_copy(ptab_hbm.at[pidx.at[pl.ds(i * C, C)]], pbuf, sem).wait()

        for g in range(C // L):
            rows = jnp.int32(g * L) + lax.iota(jnp.int32, L)

            def sum_body(h, carry, rows=rows):
                acc, acc2 = carry
                cols = jnp.broadcast_to(h.astype(jnp.int32), (L,))
                x = (plsc.load_gather(wbuf, [rows, cols])
                     + plsc.load_gather(pbuf, [rows, cols]))
                plsc.store_scatter(wbuf, [rows, cols], x)
                return acc + x, acc2 + x * x

            acc, acc2 = lax.fori_loop(
                0, HIDDEN, sum_body,
                (jnp.zeros((L,), jnp.float32), jnp.zeros((L,), jnp.float32)),
                unroll=8)
            mean_v = acc * jnp.float32(1.0 / HIDDEN)
            var_v = acc2 * jnp.float32(1.0 / HIDDEN) - mean_v * mean_v
            inv = _vrsqrt(var_v + jnp.float32(EPS))

            def norm_body(h, carry, rows=rows, mean_v=mean_v, inv=inv):
                cols = jnp.broadcast_to(h.astype(jnp.int32), (L,))
                x = plsc.load_gather(wbuf, [rows, cols])
                g = plsc.load_gather(gv, [cols])
                b = plsc.load_gather(bv, [cols])
                y = (x - mean_v) * inv * g + b
                plsc.store_scatter(wbuf, [rows, cols], y)
                return carry

            lax.fori_loop(0, HIDDEN, norm_body, 0, unroll=8)
        pltpu.sync_copy(wbuf, out_hbm.at[pl.ds(base + i * C, C)])
        return carry

    lax.fori_loop(0, NCHUNK, chunk_body, 0)


def kernel(input_ids, seq_lens, position_ids, word_embeddings,
           position_embeddings, ln_gamma, ln_beta):
    del seq_lens
    return _emb_ln_kernel(input_ids, position_ids, word_embeddings,
                          position_embeddings, ln_gamma, ln_beta)


# speedup vs baseline: 2.7234x; 2.7234x over previous
import functools

import jax
import jax.numpy as jnp
from jax import lax
from jax.experimental import pallas as pl
from jax.experimental.pallas import tpu as pltpu
from jax.experimental.pallas import tpu_sc as plsc

VOCAB = 128100
HIDDEN = 768
MAX_POS = 8192
NUM_TOKENS = 16384
EPS = 1e-7

NC = 2
NS = 16
NW = NC * NS
L = 16
TOK_PER_W = NUM_TOKENS // NW
C = 16
NCHUNK = TOK_PER_W // C
NSTEP = NCHUNK // 2
NSL = HIDDEN // L
NACC = 4


def _vrsqrt(v):
    i = lax.bitcast_convert_type(v, jnp.int32)
    i = jnp.int32(0x5F3759DF) - lax.shift_right_logical(i, jnp.int32(1))
    y = lax.bitcast_convert_type(i, jnp.float32)
    half = v * jnp.float32(0.5)
    for _ in range(3):
        y = y * (jnp.float32(1.5) - half * y * y)
    return y


_mesh = plsc.VectorSubcoreMesh(core_axis_name="c", subcore_axis_name="s")


@functools.partial(
    pl.kernel,
    mesh=_mesh,
    out_type=jax.ShapeDtypeStruct((NUM_TOKENS, HIDDEN), jnp.float32),
    compiler_params=pltpu.CompilerParams(use_tc_tiling_on_sc=False,
                                         needs_layout_passes=False),
    scratch_types=[
        pltpu.VMEM((TOK_PER_W,), jnp.int32),
        pltpu.VMEM((TOK_PER_W,), jnp.int32),
        pltpu.VMEM((C, HIDDEN), jnp.float32),
        pltpu.VMEM((C, HIDDEN), jnp.float32),
        pltpu.VMEM((C, HIDDEN), jnp.float32),
        pltpu.VMEM((C, HIDDEN), jnp.float32),
        pltpu.VMEM((C, HIDDEN), jnp.float32),
        pltpu.VMEM((C, HIDDEN), jnp.float32),
        pltpu.VMEM((HIDDEN,), jnp.float32),
        pltpu.VMEM((HIDDEN,), jnp.float32),
        pltpu.VMEM((C, L), jnp.float32),
        pltpu.VMEM((C, L), jnp.float32),
        pltpu.VMEM((C,), jnp.float32),
        pltpu.VMEM((C,), jnp.float32),
        pltpu.SemaphoreType.DMA,
        pltpu.SemaphoreType.DMA,
        pltpu.SemaphoreType.DMA,
        pltpu.SemaphoreType.DMA,
    ],
)
def _emb_ln_kernel(ids_hbm, pids_hbm, wtab_hbm, ptab_hbm, g_hbm, b_hbm,
                   out_hbm, widx, pidx, wbuf0, wbuf1, pbuf0, pbuf1,
                   obuf0, obuf1, gv, bv, abuf, a2buf, mbuf, ibuf,
                   gsem0, gsem1, osem0, osem1):
    wid = lax.axis_index("s") * NC + lax.axis_index("c")
    base = wid * TOK_PER_W

    pltpu.sync_copy(ids_hbm.at[pl.ds(base, TOK_PER_W)], widx)
    pltpu.sync_copy(pids_hbm.at[pl.ds(base, TOK_PER_W)], pidx)
    pltpu.sync_copy(g_hbm, gv)
    pltpu.sync_copy(b_hbm, bv)

    bufs = ((wbuf0, pbuf0, obuf0, gsem0, osem0),
            (wbuf1, pbuf1, obuf1, gsem1, osem1))

    def issue_gathers(c, wbuf, pbuf, gsem):
        pltpu.async_copy(wtab_hbm.at[widx.at[pl.ds(c * C, C)]], wbuf, gsem)
        pltpu.async_copy(ptab_hbm.at[pidx.at[pl.ds(c * C, C)]], pbuf, gsem)

    issue_gathers(0, wbuf0, pbuf0, gsem0)
    issue_gathers(1, wbuf1, pbuf1, gsem1)

    rows = lax.iota(jnp.int32, L)

    def step(s, carry):
        for b in range(2):
            wbuf, pbuf, obuf, gsem, osem = bufs[b]
            c = 2 * s + b

            pltpu.make_async_copy(
                wtab_hbm.at[widx.at[pl.ds(c * C, C)]], wbuf, gsem).wait()
            pltpu.make_async_copy(
                ptab_hbm.at[pidx.at[pl.ds(c * C, C)]], pbuf, gsem).wait()

            @pl.when(s >= 1)
            def _wait_prev():
                pltpu.make_async_copy(
                    obuf, out_hbm.at[pl.ds(base + (c - 2) * C, C)],
                    osem).wait()

            @plsc.parallel_loop(0, C, step=1, unroll=2)
            def _pass1(t):
                accs = [jnp.zeros((L,), jnp.float32) for _ in range(NACC)]
                accs2 = [jnp.zeros((L,), jnp.float32) for _ in range(NACC)]
                for j in range(NSL):
                    x = wbuf[t, pl.ds(j * L, L)] + pbuf[t, pl.ds(j * L, L)]
                    obuf[t, pl.ds(j * L, L)] = x
                    accs[j % NACC] = accs[j % NACC] + x
                    accs2[j % NACC] = accs2[j % NACC] + x * x
                abuf[t, :] = (accs[0] + accs[1]) + (accs[2] + accs[3])
                a2buf[t, :] = (accs2[0] + accs2[1]) + (accs2[2] + accs2[3])

            @pl.when(s < NSTEP - 1)
            def _prefetch():
                issue_gathers(c + 2, wbuf, pbuf, gsem)

            s1 = jnp.zeros((L,), jnp.float32)
            s2 = jnp.zeros((L,), jnp.float32)
            for k in range(L):
                colk = jnp.broadcast_to(jnp.int32(k), (L,))
                s1 = s1 + plsc.load_gather(abuf, [rows, colk])
                s2 = s2 + plsc.load_gather(a2buf, [rows, colk])
            mean_v = s1 * jnp.float32(1.0 / HIDDEN)
            var_v = s2 * jnp.float32(1.0 / HIDDEN) - mean_v * mean_v
            inv_v = _vrsqrt(var_v + jnp.float32(EPS))
            plsc.store_scatter(mbuf, [rows], mean_v)
            plsc.store_scatter(ibuf, [rows], inv_v)

            @plsc.parallel_loop(0, C, step=2, unroll=1)
            def _pass2(t0):
                for dt in range(2):
                    t = t0 + dt
                    tt = jnp.broadcast_to(t.astype(jnp.int32), (L,))
                    mean_b = plsc.load_gather(mbuf, [tt])
                    inv_b = plsc.load_gather(ibuf, [tt])
                    for j in range(NSL):
                        x = obuf[t, pl.ds(j * L, L)]
                        g = gv[pl.ds(j * L, L)]
                        bb = bv[pl.ds(j * L, L)]
                        obuf[t, pl.ds(j * L, L)] = \
                            (x - mean_b) * inv_b * g + bb

            pltpu.async_copy(obuf, out_hbm.at[pl.ds(base + c * C, C)], osem)
        return carry

    lax.fori_loop(0, NSTEP, step, 0)

    pltpu.make_async_copy(
        obuf0, out_hbm.at[pl.ds(base + (NCHUNK - 2) * C, C)], osem0).wait()
    pltpu.make_async_copy(
        obuf1, out_hbm.at[pl.ds(base + (NCHUNK - 1) * C, C)], osem1).wait()


def kernel(input_ids, seq_lens, position_ids, word_embeddings,
           position_embeddings, ln_gamma, ln_beta):
    del seq_lens
    return _emb_ln_kernel(input_ids, position_ids, word_embeddings,
                          position_embeddings, ln_gamma, ln_beta)


# speedup vs baseline: 6.7027x; 2.4611x over previous
import functools

import jax
import jax.numpy as jnp
from jax import lax
from jax.experimental import pallas as pl
from jax.experimental.pallas import tpu as pltpu
from jax.experimental.pallas import tpu_sc as plsc

VOCAB = 128100
HIDDEN = 768
MAX_POS = 8192
NUM_TOKENS = 16384
EPS = 1e-7

NC = 2
NS = 16
NW = NC * NS
L = 16
TOK_PER_W = NUM_TOKENS // NW
C = 16
NCHUNK = TOK_PER_W // C
NSTEP = NCHUNK // 2
NSL = HIDDEN // L
NACC = 4


def _vrsqrt(v):
    i = lax.bitcast_convert_type(v, jnp.int32)
    i = jnp.int32(0x5F3759DF) - lax.shift_right_logical(i, jnp.int32(1))
    y = lax.bitcast_convert_type(i, jnp.float32)
    half = v * jnp.float32(0.5)
    for _ in range(3):
        y = y * (jnp.float32(1.5) - half * y * y)
    return y


_mesh = plsc.VectorSubcoreMesh(core_axis_name="c", subcore_axis_name="s")


@functools.partial(
    pl.kernel,
    mesh=_mesh,
    out_type=jax.ShapeDtypeStruct((NUM_TOKENS, HIDDEN), jnp.float32),
    compiler_params=pltpu.CompilerParams(needs_layout_passes=False),
    scratch_types=[
        pltpu.VMEM((TOK_PER_W,), jnp.int32),
        pltpu.VMEM((TOK_PER_W,), jnp.int32),
        pltpu.VMEM((C, HIDDEN), jnp.float32),
        pltpu.VMEM((C, HIDDEN), jnp.float32),
        pltpu.VMEM((C, HIDDEN), jnp.float32),
        pltpu.VMEM((C, HIDDEN), jnp.float32),
        pltpu.VMEM((C, HIDDEN), jnp.float32),
        pltpu.VMEM((C, HIDDEN), jnp.float32),
        pltpu.VMEM((HIDDEN,), jnp.float32),
        pltpu.VMEM((HIDDEN,), jnp.float32),
        pltpu.VMEM((C * L,), jnp.float32),
        pltpu.VMEM((C * L,), jnp.float32),
        pltpu.VMEM((C,), jnp.float32),
        pltpu.VMEM((C,), jnp.float32),
        pltpu.SemaphoreType.DMA,
        pltpu.SemaphoreType.DMA,
        pltpu.SemaphoreType.DMA,
        pltpu.SemaphoreType.DMA,
    ],
)
def _emb_ln_kernel(ids_hbm, pids_hbm, wtab_hbm, ptab_hbm, g_hbm, b_hbm,
                   out_hbm, widx, pidx, wbuf0, wbuf1, pbuf0, pbuf1,
                   obuf0, obuf1, gv, bv, abuf, a2buf, mbuf, ibuf,
                   gsem0, gsem1, osem0, osem1):
    wid = lax.axis_index("s") * NC + lax.axis_index("c")
    base = wid * TOK_PER_W

    pltpu.sync_copy(ids_hbm.at[pl.ds(base, TOK_PER_W)], widx)
    pltpu.sync_copy(pids_hbm.at[pl.ds(base, TOK_PER_W)], pidx)
    pltpu.sync_copy(g_hbm, gv)
    pltpu.sync_copy(b_hbm, bv)

    bufs = ((wbuf0, pbuf0, obuf0, gsem0, osem0),
            (wbuf1, pbuf1, obuf1, gsem1, osem1))

    def issue_gathers(c, wbuf, pbuf, gsem):
        pltpu.async_copy(wtab_hbm.at[widx.at[pl.ds(c * C, C)]], wbuf, gsem)
        pltpu.async_copy(ptab_hbm.at[pidx.at[pl.ds(c * C, C)]], pbuf, gsem)

    issue_gathers(0, wbuf0, pbuf0, gsem0)
    issue_gathers(1, wbuf1, pbuf1, gsem1)

    rows = lax.iota(jnp.int32, L)

    def step(s, carry):
        for b in range(2):
            wbuf, pbuf, obuf, gsem, osem = bufs[b]
            c = 2 * s + b

            pltpu.make_async_copy(
                wtab_hbm.at[widx.at[pl.ds(c * C, C)]], wbuf, gsem).wait()
            pltpu.make_async_copy(
                ptab_hbm.at[pidx.at[pl.ds(c * C, C)]], pbuf, gsem).wait()

            @pl.when(s >= 1)
            def _wait_prev():
                pltpu.make_async_copy(
                    obuf, out_hbm.at[pl.ds(base + (c - 2) * C, C)],
                    osem).wait()

            @plsc.parallel_loop(0, C, step=1, unroll=2)
            def _pass1(t):
                accs = [jnp.zeros((L,), jnp.float32) for _ in range(NACC)]
                accs2 = [jnp.zeros((L,), jnp.float32) for _ in range(NACC)]
                for j in range(NSL):
                    x = wbuf[t, pl.ds(j * L, L)] + pbuf[t, pl.ds(j * L, L)]
                    obuf[t, pl.ds(j * L, L)] = x
                    accs[j % NACC] = accs[j % NACC] + x
                    accs2[j % NACC] = accs2[j % NACC] + x * x
                abuf[pl.ds(t * L, L)] = (accs[0] + accs[1]) + (accs[2] + accs[3])
                a2buf[pl.ds(t * L, L)] = (accs2[0] + accs2[1]) + (accs2[2] + accs2[3])

            @pl.when(s < NSTEP - 1)
            def _prefetch():
                issue_gathers(c + 2, wbuf, pbuf, gsem)

            s1 = jnp.zeros((L,), jnp.float32)
            s2 = jnp.zeros((L,), jnp.float32)
            rowsL = rows * jnp.int32(L)
            for k in range(L):
                s1 = s1 + plsc.load_gather(abuf, [rowsL + jnp.int32(k)])
                s2 = s2 + plsc.load_gather(a2buf, [rowsL + jnp.int32(k)])
            mean_v = s1 * jnp.float32(1.0 / HIDDEN)
            var_v = s2 * jnp.float32(1.0 / HIDDEN) - mean_v * mean_v
            inv_v = _vrsqrt(var_v + jnp.float32(EPS))
            plsc.store_scatter(mbuf, [rows], mean_v)
            plsc.store_scatter(ibuf, [rows], inv_v)

            @plsc.parallel_loop(0, C, step=2, unroll=1)
            def _pass2(t0):
                for dt in range(2):
                    t = t0 + dt
                    tt = jnp.broadcast_to(t.astype(jnp.int32), (L,))
                    mean_b = plsc.load_gather(mbuf, [tt])
                    inv_b = plsc.load_gather(ibuf, [tt])
                    for j in range(NSL):
                        x = obuf[t, pl.ds(j * L, L)]
                        g = gv[pl.ds(j * L, L)]
                        bb = bv[pl.ds(j * L, L)]
                        obuf[t, pl.ds(j * L, L)] = \
                            (x - mean_b) * inv_b * g + bb

            pltpu.async_copy(obuf, out_hbm.at[pl.ds(base + c * C, C)], osem)
        return carry

    lax.fori_loop(0, NSTEP, step, 0)

    pltpu.make_async_copy(
        obuf0, out_hbm.at[pl.ds(base + (NCHUNK - 2) * C, C)], osem0).wait()
    pltpu.make_async_copy(
        obuf1, out_hbm.at[pl.ds(base + (NCHUNK - 1) * C, C)], osem1).wait()


def kernel(input_ids, seq_lens, position_ids, word_embeddings,
           position_embeddings, ln_gamma, ln_beta):
    del seq_lens
    return _emb_ln_kernel(input_ids, position_ids, word_embeddings,
                          position_embeddings, ln_gamma, ln_beta)


# speedup vs baseline: 15.7884x; 2.3555x over previous
import functools

import jax
import jax.numpy as jnp
from jax import lax
from jax.experimental import pallas as pl
from jax.experimental.pallas import tpu as pltpu
from jax.experimental.pallas import tpu_sc as plsc

VOCAB = 128100
HIDDEN = 768
MAX_POS = 8192
NUM_TOKENS = 16384
EPS = 1e-7

NC = 2
NS = 16
NW = NC * NS
L = 16
TOK_PER_W = NUM_TOKENS // NW
C = 16
NCHUNK = TOK_PER_W // C
NSTEP = NCHUNK // 2
NSL = HIDDEN // L

BT = 1024


_mesh = plsc.VectorSubcoreMesh(core_axis_name="c", subcore_axis_name="s")


@functools.partial(
    pl.kernel,
    mesh=_mesh,
    out_type=jax.ShapeDtypeStruct((NUM_TOKENS, HIDDEN), jnp.float32),
    compiler_params=pltpu.CompilerParams(needs_layout_passes=False),
    scratch_types=[
        pltpu.VMEM((TOK_PER_W,), jnp.int32),
        pltpu.VMEM((TOK_PER_W,), jnp.int32),
        pltpu.VMEM((C, HIDDEN), jnp.float32),
        pltpu.VMEM((C, HIDDEN), jnp.float32),
        pltpu.VMEM((C, HIDDEN), jnp.float32),
        pltpu.VMEM((C, HIDDEN), jnp.float32),
        pltpu.VMEM((C, HIDDEN), jnp.float32),
        pltpu.VMEM((C, HIDDEN), jnp.float32),
        pltpu.SemaphoreType.DMA,
        pltpu.SemaphoreType.DMA,
        pltpu.SemaphoreType.DMA,
        pltpu.SemaphoreType.DMA,
    ],
)
def _gather_add_kernel(ids_hbm, pids_hbm, wtab_hbm, ptab_hbm,
                       out_hbm, widx, pidx, wbuf0, wbuf1, pbuf0, pbuf1,
                       obuf0, obuf1, gsem0, gsem1, osem0, osem1):
    wid = lax.axis_index("s") * NC + lax.axis_index("c")
    base = wid * TOK_PER_W

    pltpu.sync_copy(ids_hbm.at[pl.ds(base, TOK_PER_W)], widx)
    pltpu.sync_copy(pids_hbm.at[pl.ds(base, TOK_PER_W)], pidx)

    bufs = ((wbuf0, pbuf0, obuf0, gsem0, osem0),
            (wbuf1, pbuf1, obuf1, gsem1, osem1))

    def issue_gathers(c, wbuf, pbuf, gsem):
        pltpu.async_copy(wtab_hbm.at[widx.at[pl.ds(c * C, C)]], wbuf, gsem)
        pltpu.async_copy(ptab_hbm.at[pidx.at[pl.ds(c * C, C)]], pbuf, gsem)

    issue_gathers(0, wbuf0, pbuf0, gsem0)
    issue_gathers(1, wbuf1, pbuf1, gsem1)

    def step(s, carry):
        for b in range(2):
            wbuf, pbuf, obuf, gsem, osem = bufs[b]
            c = 2 * s + b

            pltpu.make_async_copy(
                wtab_hbm.at[widx.at[pl.ds(c * C, C)]], wbuf, gsem).wait()
            pltpu.make_async_copy(
                ptab_hbm.at[pidx.at[pl.ds(c * C, C)]], pbuf, gsem).wait()

            @pl.when(s >= 1)
            def _wait_prev():
                pltpu.make_async_copy(
                    obuf, out_hbm.at[pl.ds(base + (c - 2) * C, C)],
                    osem).wait()

            @plsc.parallel_loop(0, C, step=1, unroll=2)
            def _add(t):
                for j in range(NSL):
                    obuf[t, pl.ds(j * L, L)] = (
                        wbuf[t, pl.ds(j * L, L)] + pbuf[t, pl.ds(j * L, L)])

            @pl.when(s < NSTEP - 1)
            def _prefetch():
                issue_gathers(c + 2, wbuf, pbuf, gsem)

            pltpu.async_copy(obuf, out_hbm.at[pl.ds(base + c * C, C)], osem)
        return carry

    lax.fori_loop(0, NSTEP, step, 0)

    pltpu.make_async_copy(
        obuf0, out_hbm.at[pl.ds(base + (NCHUNK - 2) * C, C)], osem0).wait()
    pltpu.make_async_copy(
        obuf1, out_hbm.at[pl.ds(base + (NCHUNK - 1) * C, C)], osem1).wait()


def _ln_body(x_ref, g_ref, b_ref, o_ref):
    x = x_ref[...]
    mean = jnp.mean(x, axis=-1, keepdims=True)
    xc = x - mean
    var = jnp.mean(xc * xc, axis=-1, keepdims=True)
    inv = lax.rsqrt(var + jnp.float32(EPS))
    o_ref[...] = (xc * inv) * g_ref[...][None, :] + b_ref[...][None, :]


_ln_tc = pl.pallas_call(
    _ln_body,
    grid=(NUM_TOKENS // BT,),
    in_specs=[
        pl.BlockSpec((BT, HIDDEN), lambda i: (i, 0)),
        pl.BlockSpec((HIDDEN,), lambda i: (0,)),
        pl.BlockSpec((HIDDEN,), lambda i: (0,)),
    ],
    out_specs=pl.BlockSpec((BT, HIDDEN), lambda i: (i, 0)),
    out_shape=jax.ShapeDtypeStruct((NUM_TOKENS, HIDDEN), jnp.float32),
)


def kernel(input_ids, seq_lens, position_ids, word_embeddings,
           position_embeddings, ln_gamma, ln_beta):
    del seq_lens
    summed = _gather_add_kernel(input_ids, position_ids, word_embeddings,
                                position_embeddings)
    return _ln_tc(summed, ln_gamma, ln_beta)
